# Initial kernel scaffold; baseline (speedup 1.0000x reference)
import jax
import jax.numpy as jnp
from jax.experimental import pallas as pl


def kernel(u2e, v2e, W1, b1, nodes, nodes_target, history_u, uv):
    raise NotImplementedError("write your pallas kernel here")


# speedup vs baseline: 1.5538x; 1.5538x over previous
import functools

import jax
import jax.numpy as jnp
from jax import lax
from jax.experimental import pallas as pl
from jax.experimental.pallas import tpu as pltpu
from jax.experimental.pallas import tpu_sc as plsc

NUM_USERS = 100000
NUM_ITEMS = 100000
EMBED = 64
B = 4096
HIST = 50
HIST_PAD = 56
HIST_COLS = 64

NC = 2
NS = 16
NW = NC * NS
NPW = B // NW


def _sc_body(u2e_hbm, v2e_hbm, nodes_hbm, tgt_hbm, hist_hbm,
             self_out, tgt_out, neigh_out,
             nodes_v, tgtidx_v, hist_v, self_v, tgtrows_v, rows_v, neigh_v,
             sem_rows, sem_self, sem_tgt, sem_hist):
    wid = lax.axis_index("s") * NC + lax.axis_index("c")
    base = wid * NPW

    pltpu.sync_copy(nodes_hbm.at[pl.ds(base, NPW)], nodes_v)
    pltpu.sync_copy(tgt_hbm.at[pl.ds(base, NPW)], tgtidx_v)

    c_self = pltpu.async_copy(u2e_hbm.at[nodes_v], self_v, sem_self)
    c_tgt = pltpu.async_copy(v2e_hbm.at[tgtidx_v], tgtrows_v, sem_tgt)
    c_hist = pltpu.async_copy(hist_hbm.at[nodes_v], hist_v, sem_hist)
    c_hist.wait()

    def node_body(i, carry):
        pltpu.async_copy(v2e_hbm.at[hist_v.at[i, pl.ds(0, HIST_PAD)]],
                         rows_v, sem_rows).wait()

        def acc_body(j, accs):
            a0, a1, a2, a3 = accs
            return (a0 + rows_v[j, pl.ds(0, 16)],
                    a1 + rows_v[j, pl.ds(16, 16)],
                    a2 + rows_v[j, pl.ds(32, 16)],
                    a3 + rows_v[j, pl.ds(48, 16)])

        z = jnp.zeros((16,), jnp.float32)
        a0, a1, a2, a3 = lax.fori_loop(0, HIST, acc_body, (z, z, z, z))
        neigh_v[i, pl.ds(0, 16)] = a0
        neigh_v[i, pl.ds(16, 16)] = a1
        neigh_v[i, pl.ds(32, 16)] = a2
        neigh_v[i, pl.ds(48, 16)] = a3
        return carry

    lax.fori_loop(0, NPW, node_body, 0)

    c_self.wait()
    c_tgt.wait()
    pltpu.sync_copy(self_v, self_out.at[pl.ds(base, NPW)])
    pltpu.sync_copy(tgtrows_v, tgt_out.at[pl.ds(base, NPW)])
    pltpu.sync_copy(neigh_v, neigh_out.at[pl.ds(base, NPW)])


@jax.jit
def _sc_gather(u2e, v2e, nodes, nodes_target, history_u):
    mesh = plsc.VectorSubcoreMesh(core_axis_name="c", subcore_axis_name="s")
    f32 = jnp.float32
    out_type = (jax.ShapeDtypeStruct((B, EMBED), f32),
                jax.ShapeDtypeStruct((B, EMBED), f32),
                jax.ShapeDtypeStruct((B, EMBED), f32))
    scratch = [
        pltpu.VMEM((NPW,), jnp.int32),
        pltpu.VMEM((NPW,), jnp.int32),
        pltpu.VMEM((NPW, HIST_COLS), jnp.int32),
        pltpu.VMEM((NPW, EMBED), f32),
        pltpu.VMEM((NPW, EMBED), f32),
        pltpu.VMEM((HIST_PAD, EMBED), f32),
        pltpu.VMEM((NPW, EMBED), f32),
        pltpu.SemaphoreType.DMA,
        pltpu.SemaphoreType.DMA,
        pltpu.SemaphoreType.DMA,
        pltpu.SemaphoreType.DMA,
    ]
    return pl.kernel(
        _sc_body,
        out_type=out_type,
        mesh=mesh,
        scratch_types=scratch,
        compiler_params=pltpu.CompilerParams(use_tc_tiling_on_sc=False),
    )(u2e, v2e, nodes, nodes_target, history_u)


def _tc_body(self_ref, tgt_ref, neigh_ref, w1_ref, b1_ref, out_ref):
    w = w1_ref[...]
    b = b1_ref[...]
    cf = jnp.concatenate([self_ref[...], tgt_ref[...]], axis=1)
    h1 = lax.dot_general(cf, w, (((1,), (1,)), ((), ())),
                         preferred_element_type=jnp.float32)
    h1 = jnp.maximum(h1 + b, 0.0)
    neigh = neigh_ref[...] * (1.0 / HIST)
    c2 = jnp.concatenate([h1, neigh], axis=1)
    h2 = lax.dot_general(c2, w, (((1,), (1,)), ((), ())),
                         preferred_element_type=jnp.float32)
    out_ref[...] = jnp.maximum(h2 + b, 0.0)


@jax.jit
def _tc_mlp(self_feats, target_feats, neigh_sum, W1, b1):
    return pl.pallas_call(
        _tc_body,
        out_shape=jax.ShapeDtypeStruct((B, EMBED), jnp.float32),
    )(self_feats, target_feats, neigh_sum, W1, b1.reshape(1, EMBED))


def kernel(u2e, v2e, W1, b1, nodes, nodes_target, history_u, uv):
    del uv
    nodes = nodes.astype(jnp.int32)
    nodes_target = nodes_target.astype(jnp.int32)
    history_u = history_u.astype(jnp.int32)
    history_u = jnp.pad(history_u, ((0, 0), (0, HIST_COLS - HIST)))
    self_feats, target_feats, neigh_sum = _sc_gather(
        u2e, v2e, nodes, nodes_target, history_u)
    return _tc_mlp(self_feats, target_feats, neigh_sum, W1, b1)
